# Initial kernel scaffold; baseline (speedup 1.0000x reference)
import jax
import jax.numpy as jnp
from jax.experimental import pallas as pl


def kernel(x, embedding):
    raise NotImplementedError("write your pallas kernel here")


# speedup vs baseline: 1.2951x; 1.2951x over previous
import functools

import jax
import jax.numpy as jnp
from jax import lax
from jax.experimental import pallas as pl
from jax.experimental.pallas import tpu as pltpu
from jax.experimental.pallas import tpu_sc as plsc

N_E = 8192
D = 256
NTOK = 32768

TB = 256
NBLK = NTOK // TB


def _dist_argmin_body(xb_ref, et_ref, idx_ref, minsum_ref, esq_ref):
    i = pl.program_id(0)

    @pl.when(i == 0)
    def _init():
        et0 = et_ref[...]
        esq_ref[...] = jnp.sum(et0 * et0, axis=0, keepdims=True)
        minsum_ref[...] = jnp.zeros_like(minsum_ref)

    xb = xb_ref[...]
    xsq = jnp.sum(xb * xb, axis=1, keepdims=True)
    t = esq_ref[...] + xsq
    mm = lax.dot_general(xb * (-2.0), et_ref[...],
                         (((1,), (0,)), ((), ())),
                         preferred_element_type=jnp.float32)
    d = t + mm
    dmin = jnp.min(d, axis=1, keepdims=True)
    cols = lax.broadcasted_iota(jnp.int32, d.shape, 1)
    idx = jnp.min(jnp.where(d == dmin, cols, jnp.int32(N_E)), axis=1)
    idx_ref[...] = idx
    minsum_ref[...] += jnp.sum(dmin).reshape(1, 1)


_dist_argmin = pl.pallas_call(
    _dist_argmin_body,
    grid=(NBLK,),
    in_specs=[pl.BlockSpec((TB, D), lambda i: (i, 0)),
              pl.BlockSpec((D, N_E), lambda i: (0, 0))],
    out_specs=[pl.BlockSpec((TB,), lambda i: (i,)),
               pl.BlockSpec((1, 1), lambda i: (0, 0))],
    out_shape=[jax.ShapeDtypeStruct((NTOK,), jnp.int32),
               jax.ShapeDtypeStruct((1, 1), jnp.float32)],
    scratch_shapes=[pltpu.VMEM((1, N_E), jnp.float32)],
    compiler_params=pltpu.CompilerParams(dimension_semantics=("arbitrary",)),
)

_NC, _NS = 2, 16
_NW = _NC * _NS
_BPW = NTOK // _NW
_CH = 128
_GCH = _BPW // _CH

def _gather_count_body(e_hbm, idx_hbm, out_hbm, counts_hbm,
                       idx_v, rows_v, ones_v, zb_v, hist_sp, sem):
    c = lax.axis_index("c")
    s = lax.axis_index("s")
    wid = c * _NS + s
    base = wid * _BPW
    pltpu.sync_copy(idx_hbm.at[pl.ds(wid * _GCH, _GCH)], idx_v)
    for j in range(_CH // 16):
        ones_v[pl.ds(j * 16, 16)] = jnp.full((16,), 1.0, jnp.float32)

    @pl.when(s == 0)
    def _zero_hist():
        def zbody(j, carry):
            zb_v[pl.ds(j * 16, 16)] = jnp.zeros((16,), jnp.float32)
            return carry
        lax.fori_loop(0, N_E // 16, zbody, 0)
        pltpu.sync_copy(zb_v, hist_sp)

    plsc.subcore_barrier()
    for g in range(_GCH):
        pltpu.sync_copy(ones_v, hist_sp.at[idx_v.at[g]], add=True)
    plsc.subcore_barrier()

    @pl.when(s == 0)
    def _write_hist():
        pltpu.sync_copy(hist_sp, counts_hbm.at[c])

    for g in range(_GCH):
        pltpu.async_copy(e_hbm.at[idx_v.at[g]], rows_v, sem).wait()
        pltpu.sync_copy(rows_v, out_hbm.at[pl.ds(base + g * _CH, _CH)])

@functools.cache
def _gather_count():
    mesh = plsc.VectorSubcoreMesh(core_axis_name="c", subcore_axis_name="s",
                                  num_cores=_NC, num_subcores=_NS)
    return pl.kernel(
        _gather_count_body,
        out_type=[jax.ShapeDtypeStruct((NTOK, D), jnp.float32),
                  jax.ShapeDtypeStruct((_NC, N_E), jnp.float32)],
        mesh=mesh,
        scratch_types=[pltpu.VMEM((_GCH, _CH), jnp.int32),
                       pltpu.VMEM((_CH, D), jnp.float32),
                       pltpu.VMEM((_CH,), jnp.float32),
                       pltpu.VMEM((N_E,), jnp.float32),
                       pltpu.VMEM_SHARED((N_E,), jnp.float32),
                       pltpu.SemaphoreType.DMA],
    )


def _stats_body(counts_ref, minsum_ref, used_ref, perp_ref, commit_ref):
    counts = jnp.sum(counts_ref[...], axis=0, keepdims=True)
    used_ref[...] = jnp.sum((counts >= 1.0).astype(jnp.float32),
                            axis=1, keepdims=True)
    avg = counts * (1.0 / NTOK)
    ent = jnp.sum(avg * jnp.log(avg + 1e-10), axis=1, keepdims=True)
    perp_ref[...] = jnp.exp(-ent)
    commit_ref[...] = minsum_ref[...] * (1.0 / (NTOK * D))


_stats = pl.pallas_call(
    _stats_body,
    out_shape=[jax.ShapeDtypeStruct((1, 1), jnp.float32),
               jax.ShapeDtypeStruct((1, 1), jnp.float32),
               jax.ShapeDtypeStruct((1, 1), jnp.float32)],
)


def kernel(x, embedding):
    bsz, tsz, _ = x.shape
    x_flat = x.reshape(NTOK, D)
    et = embedding.T
    idx, minsum = _dist_argmin(x_flat, et)
    idx2 = idx.reshape(NTOK // _CH, _CH)
    quant, counts = _gather_count()(embedding, idx2)
    used, perp, commit = _stats(counts, minsum)
    quantized_st = quant.reshape(x.shape)
    inds = idx.reshape(bsz, tsz, 1)
    return (quantized_st, inds, used.reshape(()), perp.reshape(()),
            commit.reshape(()))


# speedup vs baseline: 1.4819x; 1.1443x over previous
import functools

import jax
import jax.numpy as jnp
from jax import lax
from jax.experimental import pallas as pl
from jax.experimental.pallas import tpu as pltpu
from jax.experimental.pallas import tpu_sc as plsc

N_E = 8192
D = 256
NTOK = 32768

TB = 1024
NBLK = NTOK // TB


def _dist_argmin_body(xb_ref, et_ref, idx_ref, minsum_ref, esq_ref, colsf_ref):
    i = pl.program_id(0)

    @pl.when(i == 0)
    def _init():
        et0 = et_ref[...]
        esq_ref[...] = jnp.sum(et0 * et0, axis=0, keepdims=True)
        colsf_ref[...] = lax.broadcasted_iota(
            jnp.int32, (1, N_E), 1).astype(jnp.float32)
        minsum_ref[...] = jnp.zeros_like(minsum_ref)

    xb = xb_ref[...]
    xsq = jnp.sum(xb * xb, axis=1, keepdims=True)
    t = esq_ref[...] + xsq
    mm = lax.dot_general(xb * (-2.0), et_ref[...],
                         (((1,), (0,)), ((), ())),
                         preferred_element_type=jnp.float32)
    d = t + mm
    dmin = jnp.min(d, axis=1, keepdims=True)
    idxf = jnp.min(jnp.where(d == dmin, colsf_ref[...], jnp.float32(N_E)),
                   axis=1)
    idx_ref[...] = idxf.astype(jnp.int32)
    minsum_ref[...] += jnp.sum(dmin).reshape(1, 1)


_dist_argmin = pl.pallas_call(
    _dist_argmin_body,
    grid=(NBLK,),
    in_specs=[pl.BlockSpec((TB, D), lambda i: (i, 0)),
              pl.BlockSpec((D, N_E), lambda i: (0, 0))],
    out_specs=[pl.BlockSpec((TB,), lambda i: (i,)),
               pl.BlockSpec((1, 1), lambda i: (0, 0))],
    out_shape=[jax.ShapeDtypeStruct((NTOK,), jnp.int32),
               jax.ShapeDtypeStruct((1, 1), jnp.float32)],
    scratch_shapes=[pltpu.VMEM((1, N_E), jnp.float32),
                    pltpu.VMEM((1, N_E), jnp.float32)],
    compiler_params=pltpu.CompilerParams(dimension_semantics=("arbitrary",)),
)

_NC, _NS = 2, 16
_NW = _NC * _NS
_BPW = NTOK // _NW
_CH = 128
_GCH = _BPW // _CH

def _gather_count_body(e_hbm, idx_hbm, out_hbm, counts_hbm,
                       idx_v, rows_v, ones_v, zb_v, hist_sp,
                       sem0, sem1, sem_h):
    c = lax.axis_index("c")
    s = lax.axis_index("s")
    wid = c * _NS + s
    base = wid * _BPW
    sems = (sem0, sem1)
    pltpu.sync_copy(idx_hbm.at[pl.ds(wid * _GCH, _GCH)], idx_v)
    for j in range(_CH // 16):
        ones_v[pl.ds(j * 16, 16)] = jnp.full((16,), 1.0, jnp.float32)

    @pl.when(s == 0)
    def _zero_hist():
        def zbody(j, carry):
            zb_v[pl.ds(j * 16, 16)] = jnp.zeros((16,), jnp.float32)
            return carry
        lax.fori_loop(0, N_E // 16, zbody, 0)
        pltpu.sync_copy(zb_v, hist_sp)

    plsc.subcore_barrier()
    hcps = [pltpu.async_copy(ones_v, hist_sp.at[idx_v.at[g]], sem_h,
                             add=True)
            for g in range(_GCH)]

    cp = pltpu.async_copy(e_hbm.at[idx_v.at[0]], rows_v.at[0], sems[0])
    for g in range(_GCH):
        nxt = None
        if g + 1 < _GCH:
            nxt = pltpu.async_copy(e_hbm.at[idx_v.at[g + 1]],
                                   rows_v.at[(g + 1) % 2], sems[(g + 1) % 2])
        cp.wait()
        pltpu.sync_copy(rows_v.at[g % 2], out_hbm.at[pl.ds(base + g * _CH,
                                                           _CH)])
        cp = nxt

    for h in hcps:
        h.wait()
    plsc.subcore_barrier()

    @pl.when(s == 0)
    def _write_hist():
        pltpu.sync_copy(hist_sp, counts_hbm.at[c])

@functools.cache
def _gather_count():
    mesh = plsc.VectorSubcoreMesh(core_axis_name="c", subcore_axis_name="s",
                                  num_cores=_NC, num_subcores=_NS)
    return pl.kernel(
        _gather_count_body,
        out_type=[jax.ShapeDtypeStruct((NTOK, D), jnp.float32),
                  jax.ShapeDtypeStruct((_NC, N_E), jnp.float32)],
        mesh=mesh,
        scratch_types=[pltpu.VMEM((_GCH, _CH), jnp.int32),
                       pltpu.VMEM((2, _CH, D), jnp.float32),
                       pltpu.VMEM((_CH,), jnp.float32),
                       pltpu.VMEM((N_E,), jnp.float32),
                       pltpu.VMEM_SHARED((N_E,), jnp.float32),
                       pltpu.SemaphoreType.DMA,
                       pltpu.SemaphoreType.DMA,
                       pltpu.SemaphoreType.DMA],
    )


def _stats_body(counts_ref, minsum_ref, used_ref, perp_ref, commit_ref):
    counts = jnp.sum(counts_ref[...], axis=0, keepdims=True)
    used_ref[...] = jnp.sum((counts >= 1.0).astype(jnp.float32),
                            axis=1, keepdims=True)
    avg = counts * (1.0 / NTOK)
    ent = jnp.sum(avg * jnp.log(avg + 1e-10), axis=1, keepdims=True)
    perp_ref[...] = jnp.exp(-ent)
    commit_ref[...] = minsum_ref[...] * (1.0 / (NTOK * D))


_stats = pl.pallas_call(
    _stats_body,
    out_shape=[jax.ShapeDtypeStruct((1, 1), jnp.float32),
               jax.ShapeDtypeStruct((1, 1), jnp.float32),
               jax.ShapeDtypeStruct((1, 1), jnp.float32)],
)


def kernel(x, embedding):
    bsz, tsz, _ = x.shape
    x_flat = x.reshape(NTOK, D)
    et = embedding.T
    idx, minsum = _dist_argmin(x_flat, et)
    idx2 = idx.reshape(NTOK // _CH, _CH)
    quant, counts = _gather_count()(embedding, idx2)
    used, perp, commit = _stats(counts, minsum)
    quantized_st = quant.reshape(x.shape)
    inds = idx.reshape(bsz, tsz, 1)
    return (quantized_st, inds, used.reshape(()), perp.reshape(()),
            commit.reshape(()))
